# baseline (speedup 1.0000x reference)
import jax
import jax.numpy as jnp
from jax import lax
from jax.experimental import pallas as pl
from jax.experimental.pallas import tpu as pltpu
from jax.experimental.pallas import tpu_sc as plsc

_N, _M, _D = 10000, 16, 256
_ROW = _M * _D
_L = 16
_NW = 32
_C = 8
_W_CNT = 313
_NBLK = (_W_CNT + _C - 1) // _C
_PAIRS = _NBLK // 2


def _sc_body(mail_hbm, out_hbm, buf, obuf, isem0, isem1, osem0, osem1):
    cid = lax.axis_index("c")
    sid = lax.axis_index("s")
    wid = sid * 2 + cid
    w_start = jnp.minimum(wid * _W_CNT, _N - _W_CNT)

    def blk_node0(i):
        return w_start + jnp.minimum(i * _C, _W_CNT - _C)

    def start_in(i, slot, sem):
        n0 = blk_node0(i)
        pltpu.async_copy(
            mail_hbm.at[pl.ds(n0 * _ROW, _C * _ROW)],
            buf.at[pl.ds(slot * (_C * _ROW), _C * _ROW)],
            sem,
        )

    def wait_in(slot, sem):
        pltpu.make_async_copy(
            mail_hbm.at[pl.ds(0, _C * _ROW)],
            buf.at[pl.ds(slot * (_C * _ROW), _C * _ROW)],
            sem,
        ).wait()

    def start_out(i, slot, sem):
        n0 = blk_node0(i)
        pltpu.async_copy(
            obuf.at[pl.ds(slot * (_C * _D), _C * _D)],
            out_hbm.at[pl.ds(n0 * _D, _C * _D)],
            sem,
        )

    def wait_out(slot, sem):
        pltpu.make_async_copy(
            obuf.at[pl.ds(slot * (_C * _D), _C * _D)],
            out_hbm.at[pl.ds(0, _C * _D)],
            sem,
        ).wait()

    def compute_block(slot):
        base = slot * (_C * _ROW)
        obase = slot * (_C * _D)

        def node_body(j, carry):
            boff = base + j * _ROW
            ooff = obase + j * _D
            for f in range(_D // _L):
                vals = [buf[pl.ds(boff + m * _D + f * _L, _L)] for m in range(_M)]
                while len(vals) > 1:
                    vals = [jnp.maximum(vals[k], vals[k + 1])
                            for k in range(0, len(vals) - 1, 2)] + (
                        [vals[-1]] if len(vals) % 2 else [])
                obuf[pl.ds(ooff + f * _L, _L)] = vals[0]
            return carry

        lax.fori_loop(0, _C, node_body, 0)

    start_in(0, 0, isem0)

    def pair_body(p, carry):
        i0 = 2 * p
        i1 = i0 + 1
        start_in(i1, 1, isem1)
        wait_in(0, isem0)

        @pl.when(p >= 1)
        def _():
            wait_out(0, osem0)

        compute_block(0)
        start_out(i0, 0, osem0)

        @pl.when(p < _PAIRS - 1)
        def _():
            start_in(i0 + 2, 0, isem0)

        wait_in(1, isem1)

        @pl.when(p >= 1)
        def _():
            wait_out(1, osem1)

        compute_block(1)
        start_out(i1, 1, osem1)
        return carry

    lax.fori_loop(0, _PAIRS, pair_body, 0)
    wait_out(0, osem0)
    wait_out(1, osem1)


def _sc_call(mail_flat):
    mesh = plsc.VectorSubcoreMesh(core_axis_name="c", subcore_axis_name="s")
    return pl.kernel(
        _sc_body,
        out_type=jax.ShapeDtypeStruct((_N * _D,), jnp.float32),
        mesh=mesh,
        scratch_types=[
            pltpu.VMEM((2 * _C * _ROW,), jnp.float32),
            pltpu.VMEM((2 * _C * _D,), jnp.float32),
            pltpu.SemaphoreType.DMA,
            pltpu.SemaphoreType.DMA,
            pltpu.SemaphoreType.DMA,
            pltpu.SemaphoreType.DMA,
        ],
    )(mail_flat)


def kernel(mailbox):
    n, m, d = mailbox.shape
    out_flat = _sc_call(mailbox.reshape(-1))
    return out_flat.reshape(n, d)


# speedup vs baseline: 2.2680x; 2.2680x over previous
import jax
import jax.numpy as jnp
from jax import lax
from jax.experimental import pallas as pl
from jax.experimental.pallas import tpu as pltpu
from jax.experimental.pallas import tpu_sc as plsc

_N, _M, _D = 10000, 16, 256
_L = 16
_NW = 32
_C = 8
_TOT_BLK = _N // _C
_NBLK = -(-_TOT_BLK // _NW)
_PAIRS = _NBLK // 2


def _sc_body(mail_hbm, out_hbm, buf, obuf, isem0, isem1, osem0, osem1):
    cid = lax.axis_index("c")
    sid = lax.axis_index("s")
    wid = sid * 2 + cid
    w_blk0 = jnp.minimum(wid * _NBLK, _TOT_BLK - _NBLK)

    def blk_node0(i):
        return (w_blk0 + i) * _C

    def start_in(i, slot, sem):
        n0 = blk_node0(i)
        pltpu.async_copy(
            mail_hbm.at[pl.ds(n0, _C)],
            buf.at[pl.ds(slot * _C, _C)],
            sem,
        )

    def wait_in(slot, sem):
        pltpu.make_async_copy(
            mail_hbm.at[pl.ds(0, _C)],
            buf.at[pl.ds(slot * _C, _C)],
            sem,
        ).wait()

    def start_out(i, slot, sem):
        n0 = blk_node0(i)
        pltpu.async_copy(
            obuf.at[pl.ds(slot * _C, _C)],
            out_hbm.at[pl.ds(n0, _C)],
            sem,
        )

    def wait_out(slot, sem):
        pltpu.make_async_copy(
            obuf.at[pl.ds(slot * _C, _C)],
            out_hbm.at[pl.ds(0, _C)],
            sem,
        ).wait()

    def compute_block(slot):
        def node_body(j, carry):
            idx = slot * _C + j
            for f in range(_D // _L):
                vals = [buf[idx, m, pl.ds(f * _L, _L)] for m in range(_M)]
                while len(vals) > 1:
                    vals = [jnp.maximum(vals[k], vals[k + 1])
                            for k in range(0, len(vals) - 1, 2)] + (
                        [vals[-1]] if len(vals) % 2 else [])
                obuf[idx, pl.ds(f * _L, _L)] = vals[0]
            return carry

        lax.fori_loop(0, _C, node_body, 0)

    start_in(0, 0, isem0)

    def pair_body(p, carry):
        i0 = 2 * p
        i1 = i0 + 1
        start_in(i1, 1, isem1)
        wait_in(0, isem0)

        @pl.when(p >= 1)
        def _():
            wait_out(0, osem0)

        compute_block(0)
        start_out(i0, 0, osem0)

        @pl.when(p < _PAIRS - 1)
        def _():
            start_in(i0 + 2, 0, isem0)

        wait_in(1, isem1)

        @pl.when(p >= 1)
        def _():
            wait_out(1, osem1)

        compute_block(1)
        start_out(i1, 1, osem1)
        return carry

    lax.fori_loop(0, _PAIRS, pair_body, 0)
    wait_out(0, osem0)
    wait_out(1, osem1)


def kernel(mailbox):
    n, m, d = mailbox.shape
    mesh = plsc.VectorSubcoreMesh(core_axis_name="c", subcore_axis_name="s")
    return pl.kernel(
        _sc_body,
        out_type=jax.ShapeDtypeStruct((n, d), jnp.float32),
        mesh=mesh,
        scratch_types=[
            pltpu.VMEM((2 * _C, _M, _D), jnp.float32),
            pltpu.VMEM((2 * _C, _D), jnp.float32),
            pltpu.SemaphoreType.DMA,
            pltpu.SemaphoreType.DMA,
            pltpu.SemaphoreType.DMA,
            pltpu.SemaphoreType.DMA,
        ],
    )(mailbox)


# speedup vs baseline: 2.7013x; 1.1910x over previous
import jax
import jax.numpy as jnp
from jax import lax
from jax.experimental import pallas as pl
from jax.experimental.pallas import tpu as pltpu
from jax.experimental.pallas import tpu_sc as plsc

_N, _M, _D = 10000, 16, 256
_L = 16
_NW = 32
_C = 8

_T = 6600
_TCB = 200

_SC_ROWS = _N - _T
_TOT_BLK = _SC_ROWS // _C
_NBLK = -(-_TOT_BLK // _NW)
_PAIRS = _NBLK // 2
assert _NBLK % 2 == 0 and _SC_ROWS % _C == 0 and _T % _TCB == 0


def _sc_body(mail_hbm, out_hbm, buf, obuf, isem0, isem1, osem0, osem1):
    cid = lax.axis_index("c")
    sid = lax.axis_index("s")
    wid = sid * 2 + cid
    w_blk0 = jnp.minimum(wid * _NBLK, _TOT_BLK - _NBLK)

    def blk_node0(i):
        return (w_blk0 + i) * _C

    def start_in(i, slot, sem):
        n0 = _T + blk_node0(i)
        pltpu.async_copy(
            mail_hbm.at[pl.ds(n0, _C)],
            buf.at[pl.ds(slot * _C, _C)],
            sem,
        )

    def wait_in(slot, sem):
        pltpu.make_async_copy(
            mail_hbm.at[pl.ds(0, _C)],
            buf.at[pl.ds(slot * _C, _C)],
            sem,
        ).wait()

    def start_out(i, slot, sem):
        n0 = blk_node0(i)
        pltpu.async_copy(
            obuf.at[pl.ds(slot * _C, _C)],
            out_hbm.at[pl.ds(n0, _C)],
            sem,
        )

    def wait_out(slot, sem):
        pltpu.make_async_copy(
            obuf.at[pl.ds(slot * _C, _C)],
            out_hbm.at[pl.ds(0, _C)],
            sem,
        ).wait()

    def compute_block(slot):
        def node_body(j, carry):
            idx = slot * _C + j
            for f in range(_D // _L):
                vals = [buf[idx, m, pl.ds(f * _L, _L)] for m in range(_M)]
                while len(vals) > 1:
                    vals = [jnp.maximum(vals[k], vals[k + 1])
                            for k in range(0, len(vals) - 1, 2)] + (
                        [vals[-1]] if len(vals) % 2 else [])
                obuf[idx, pl.ds(f * _L, _L)] = vals[0]
            return carry

        lax.fori_loop(0, _C, node_body, 0)

    start_in(0, 0, isem0)

    def pair_body(p, carry):
        i0 = 2 * p
        i1 = i0 + 1
        start_in(i1, 1, isem1)
        wait_in(0, isem0)

        @pl.when(p >= 1)
        def _():
            wait_out(0, osem0)

        compute_block(0)
        start_out(i0, 0, osem0)

        @pl.when(p < _PAIRS - 1)
        def _():
            start_in(i0 + 2, 0, isem0)

        wait_in(1, isem1)

        @pl.when(p >= 1)
        def _():
            wait_out(1, osem1)

        compute_block(1)
        start_out(i1, 1, osem1)
        return carry

    lax.fori_loop(0, _PAIRS, pair_body, 0)
    wait_out(0, osem0)
    wait_out(1, osem1)


def _sc_call(mailbox):
    mesh = plsc.VectorSubcoreMesh(core_axis_name="c", subcore_axis_name="s")
    return pl.kernel(
        _sc_body,
        out_type=jax.ShapeDtypeStruct((_SC_ROWS, _D), jnp.float32),
        mesh=mesh,
        scratch_types=[
            pltpu.VMEM((2 * _C, _M, _D), jnp.float32),
            pltpu.VMEM((2 * _C, _D), jnp.float32),
            pltpu.SemaphoreType.DMA,
            pltpu.SemaphoreType.DMA,
            pltpu.SemaphoreType.DMA,
            pltpu.SemaphoreType.DMA,
        ],
    )(mailbox)


def _tc_body(mail_ref, out_ref):
    out_ref[...] = jnp.max(mail_ref[...], axis=1)


def _tc_call(mailbox):
    return pl.pallas_call(
        _tc_body,
        grid=(_T // _TCB,),
        in_specs=[pl.BlockSpec((_TCB, _M, _D), lambda i: (i, 0, 0))],
        out_specs=pl.BlockSpec((_TCB, _D), lambda i: (i, 0)),
        out_shape=jax.ShapeDtypeStruct((_T, _D), mailbox.dtype),
    )(mailbox)


def kernel(mailbox):
    sc_out = _sc_call(mailbox)
    tc_out = _tc_call(mailbox)
    return jnp.concatenate([tc_out, sc_out], axis=0)


# speedup vs baseline: 2.8488x; 1.0546x over previous
import jax
import jax.numpy as jnp
from jax import lax
from jax.experimental import pallas as pl
from jax.experimental.pallas import tpu as pltpu
from jax.experimental.pallas import tpu_sc as plsc

_N, _M, _D = 10000, 16, 256
_L = 16
_NW = 32
_C = 8

_T = 8400
_TCB = 400

_SC_ROWS = _N - _T
_TOT_BLK = _SC_ROWS // _C
_NBLK = -(-_TOT_BLK // _NW)
_PAIRS = _NBLK // 2
_ODD = _NBLK % 2 == 1
assert _SC_ROWS % _C == 0 and _T % _TCB == 0 and _TOT_BLK >= _NBLK


def _sc_body(mail_hbm, out_hbm, buf, obuf, isem0, isem1, osem0, osem1):
    cid = lax.axis_index("c")
    sid = lax.axis_index("s")
    wid = sid * 2 + cid
    w_blk0 = jnp.minimum(wid * _NBLK, _TOT_BLK - _NBLK)

    def blk_node0(i):
        return (w_blk0 + i) * _C

    def start_in(i, slot, sem):
        n0 = _T + blk_node0(i)
        pltpu.async_copy(
            mail_hbm.at[pl.ds(n0, _C)],
            buf.at[pl.ds(slot * _C, _C)],
            sem,
        )

    def wait_in(slot, sem):
        pltpu.make_async_copy(
            mail_hbm.at[pl.ds(0, _C)],
            buf.at[pl.ds(slot * _C, _C)],
            sem,
        ).wait()

    def start_out(i, slot, sem):
        n0 = blk_node0(i)
        pltpu.async_copy(
            obuf.at[pl.ds(slot * _C, _C)],
            out_hbm.at[pl.ds(n0, _C)],
            sem,
        )

    def wait_out(slot, sem):
        pltpu.make_async_copy(
            obuf.at[pl.ds(slot * _C, _C)],
            out_hbm.at[pl.ds(0, _C)],
            sem,
        ).wait()

    def compute_block(slot):
        def node_body(j, carry):
            idx = slot * _C + j
            for f in range(_D // _L):
                vals = [buf[idx, m, pl.ds(f * _L, _L)] for m in range(_M)]
                while len(vals) > 1:
                    vals = [jnp.maximum(vals[k], vals[k + 1])
                            for k in range(0, len(vals) - 1, 2)] + (
                        [vals[-1]] if len(vals) % 2 else [])
                obuf[idx, pl.ds(f * _L, _L)] = vals[0]
            return carry

        lax.fori_loop(0, _C, node_body, 0)

    start_in(0, 0, isem0)

    def pair_body(p, carry):
        i0 = 2 * p
        i1 = i0 + 1
        start_in(i1, 1, isem1)
        wait_in(0, isem0)

        @pl.when(p >= 1)
        def _():
            wait_out(0, osem0)

        compute_block(0)
        start_out(i0, 0, osem0)

        @pl.when(jnp.logical_or(p < _PAIRS - 1, _ODD))
        def _():
            start_in(i0 + 2, 0, isem0)

        wait_in(1, isem1)

        @pl.when(p >= 1)
        def _():
            wait_out(1, osem1)

        compute_block(1)
        start_out(i1, 1, osem1)
        return carry

    lax.fori_loop(0, _PAIRS, pair_body, 0)
    if _ODD:
        wait_in(0, isem0)
        wait_out(0, osem0)
        compute_block(0)
        start_out(_NBLK - 1, 0, osem0)
    wait_out(0, osem0)
    wait_out(1, osem1)


def _sc_call(mailbox):
    mesh = plsc.VectorSubcoreMesh(core_axis_name="c", subcore_axis_name="s")
    return pl.kernel(
        _sc_body,
        out_type=jax.ShapeDtypeStruct((_SC_ROWS, _D), jnp.float32),
        mesh=mesh,
        scratch_types=[
            pltpu.VMEM((2 * _C, _M, _D), jnp.float32),
            pltpu.VMEM((2 * _C, _D), jnp.float32),
            pltpu.SemaphoreType.DMA,
            pltpu.SemaphoreType.DMA,
            pltpu.SemaphoreType.DMA,
            pltpu.SemaphoreType.DMA,
        ],
    )(mailbox)


def _tc_body(mail_ref, out_ref):
    out_ref[...] = jnp.max(mail_ref[...], axis=1)


def _tc_call(mailbox):
    return pl.pallas_call(
        _tc_body,
        grid=(_T // _TCB,),
        in_specs=[pl.BlockSpec((_TCB, _M, _D), lambda i: (i, 0, 0))],
        out_specs=pl.BlockSpec((_TCB, _D), lambda i: (i, 0)),
        out_shape=jax.ShapeDtypeStruct((_T, _D), mailbox.dtype),
    )(mailbox)


def kernel(mailbox):
    sc_out = _sc_call(mailbox)
    tc_out = _tc_call(mailbox)
    return jnp.concatenate([tc_out, sc_out], axis=0)


# speedup vs baseline: 3.1021x; 1.0889x over previous
import jax
import jax.numpy as jnp
from jax import lax
from jax.experimental import pallas as pl
from jax.experimental.pallas import tpu as pltpu
from jax.experimental.pallas import tpu_sc as plsc

_N, _M, _D = 10000, 16, 256
_L = 16
_NW = 32
_C = 8

_T = 8400
_TCB = 400

_SC_ROWS = _N - _T
_TOT_BLK = _SC_ROWS // _C
_NBLK = -(-_TOT_BLK // _NW)
_PAIRS = _NBLK // 2
_ODD = _NBLK % 2 == 1
assert _SC_ROWS % _C == 0 and _T % _TCB == 0 and _TOT_BLK >= _NBLK


def _sc_body(mail_hbm, out_hbm, buf, obuf, isem0, isem1, osem0, osem1):
    cid = lax.axis_index("c")
    sid = lax.axis_index("s")
    wid = sid * 2 + cid
    w_blk0 = jnp.minimum(wid * _NBLK, _TOT_BLK - _NBLK)

    def blk_node0(i):
        return (w_blk0 + i) * _C

    def start_in(i, slot, sem):
        n0 = _T + blk_node0(i)
        pltpu.async_copy(
            mail_hbm.at[pl.ds(n0, _C)],
            buf.at[pl.ds(slot * _C, _C)],
            sem,
        )

    def wait_in(slot, sem):
        pltpu.make_async_copy(
            mail_hbm.at[pl.ds(0, _C)],
            buf.at[pl.ds(slot * _C, _C)],
            sem,
        ).wait()

    def start_out(i, slot, sem):
        n0 = blk_node0(i)
        pltpu.async_copy(
            obuf.at[pl.ds(slot * _C, _C)],
            out_hbm.at[pl.ds(n0, _C)],
            sem,
        )

    def wait_out(slot, sem):
        pltpu.make_async_copy(
            obuf.at[pl.ds(slot * _C, _C)],
            out_hbm.at[pl.ds(0, _C)],
            sem,
        ).wait()

    def compute_block(slot):
        def node_body(j, carry):
            idx = slot * _C + j
            for f in range(_D // _L):
                vals = [buf[idx, m, pl.ds(f * _L, _L)] for m in range(_M)]
                while len(vals) > 1:
                    vals = [jnp.maximum(vals[k], vals[k + 1])
                            for k in range(0, len(vals) - 1, 2)] + (
                        [vals[-1]] if len(vals) % 2 else [])
                obuf[idx, pl.ds(f * _L, _L)] = vals[0]
            return carry

        lax.fori_loop(0, _C, node_body, 0)

    start_in(0, 0, isem0)

    def pair_body(p, carry):
        i0 = 2 * p
        i1 = i0 + 1
        start_in(i1, 1, isem1)
        wait_in(0, isem0)

        @pl.when(p >= 1)
        def _():
            wait_out(0, osem0)

        compute_block(0)
        start_out(i0, 0, osem0)

        @pl.when(jnp.logical_or(p < _PAIRS - 1, _ODD))
        def _():
            start_in(i0 + 2, 0, isem0)

        wait_in(1, isem1)

        @pl.when(p >= 1)
        def _():
            wait_out(1, osem1)

        compute_block(1)
        start_out(i1, 1, osem1)
        return carry

    lax.fori_loop(0, _PAIRS, pair_body, 0)
    if _ODD:
        wait_in(0, isem0)
        wait_out(0, osem0)
        compute_block(0)
        start_out(_NBLK - 1, 0, osem0)
    wait_out(0, osem0)
    wait_out(1, osem1)


def _sc_call(mailbox):
    mesh = plsc.VectorSubcoreMesh(core_axis_name="c", subcore_axis_name="s")
    return pl.kernel(
        _sc_body,
        out_type=jax.ShapeDtypeStruct((_SC_ROWS, _D), jnp.float32),
        mesh=mesh,
        scratch_types=[
            pltpu.VMEM((2 * _C, _M, _D), jnp.float32),
            pltpu.VMEM((2 * _C, _D), jnp.float32),
            pltpu.SemaphoreType.DMA,
            pltpu.SemaphoreType.DMA,
            pltpu.SemaphoreType.DMA,
            pltpu.SemaphoreType.DMA,
        ],
    )(mailbox)


def _tc_body(mail_ref, out_ref):
    out_ref[...] = jnp.max(mail_ref[...], axis=1)


def _tc_call(mailbox):
    return pl.pallas_call(
        _tc_body,
        grid=(_T // _TCB,),
        in_specs=[pl.BlockSpec((_TCB, _M, _D), lambda i: (i, 0, 0))],
        out_specs=pl.BlockSpec((_TCB, _D), lambda i: (i, 0)),
        out_shape=jax.ShapeDtypeStruct((_N, _D), mailbox.dtype),
    )(mailbox)


def kernel(mailbox):
    sc_out = _sc_call(mailbox)
    tc_out = _tc_call(mailbox)
    return lax.dynamic_update_slice(tc_out, sc_out, (_T, 0))


# speedup vs baseline: 3.1750x; 1.0235x over previous
import jax
import jax.numpy as jnp
from jax import lax
from jax.experimental import pallas as pl
from jax.experimental.pallas import tpu as pltpu
from jax.experimental.pallas import tpu_sc as plsc

_N, _M, _D = 10000, 16, 256
_L = 16
_NW = 32
_C = 8

_T = 9600
_TCB = 400

_SC_ROWS = _N - _T
_TOT_BLK = _SC_ROWS // _C
_NBLK = -(-_TOT_BLK // _NW)
_PAIRS = _NBLK // 2
_ODD = _NBLK % 2 == 1
assert _SC_ROWS % _C == 0 and _T % _TCB == 0 and _TOT_BLK >= _NBLK


def _sc_body(mail_hbm, out_hbm, buf, obuf, isem0, isem1, osem0, osem1):
    cid = lax.axis_index("c")
    sid = lax.axis_index("s")
    wid = sid * 2 + cid
    w_blk0 = jnp.minimum(wid * _NBLK, _TOT_BLK - _NBLK)

    def blk_node0(i):
        return (w_blk0 + i) * _C

    def start_in(i, slot, sem):
        n0 = _T + blk_node0(i)
        pltpu.async_copy(
            mail_hbm.at[pl.ds(n0, _C)],
            buf.at[pl.ds(slot * _C, _C)],
            sem,
        )

    def wait_in(slot, sem):
        pltpu.make_async_copy(
            mail_hbm.at[pl.ds(0, _C)],
            buf.at[pl.ds(slot * _C, _C)],
            sem,
        ).wait()

    def start_out(i, slot, sem):
        n0 = blk_node0(i)
        pltpu.async_copy(
            obuf.at[pl.ds(slot * _C, _C)],
            out_hbm.at[pl.ds(n0, _C)],
            sem,
        )

    def wait_out(slot, sem):
        pltpu.make_async_copy(
            obuf.at[pl.ds(slot * _C, _C)],
            out_hbm.at[pl.ds(0, _C)],
            sem,
        ).wait()

    def compute_block(slot):
        def node_body(j, carry):
            idx = slot * _C + j
            for f in range(_D // _L):
                vals = [buf[idx, m, pl.ds(f * _L, _L)] for m in range(_M)]
                while len(vals) > 1:
                    vals = [jnp.maximum(vals[k], vals[k + 1])
                            for k in range(0, len(vals) - 1, 2)] + (
                        [vals[-1]] if len(vals) % 2 else [])
                obuf[idx, pl.ds(f * _L, _L)] = vals[0]
            return carry

        lax.fori_loop(0, _C, node_body, 0)

    start_in(0, 0, isem0)

    def pair_body(p, carry):
        i0 = 2 * p
        i1 = i0 + 1
        start_in(i1, 1, isem1)
        wait_in(0, isem0)

        @pl.when(p >= 1)
        def _():
            wait_out(0, osem0)

        compute_block(0)
        start_out(i0, 0, osem0)

        @pl.when(jnp.logical_or(p < _PAIRS - 1, _ODD))
        def _():
            start_in(i0 + 2, 0, isem0)

        wait_in(1, isem1)

        @pl.when(p >= 1)
        def _():
            wait_out(1, osem1)

        compute_block(1)
        start_out(i1, 1, osem1)
        return carry

    lax.fori_loop(0, _PAIRS, pair_body, 0)
    if _ODD:
        wait_in(0, isem0)
        wait_out(0, osem0)
        compute_block(0)
        start_out(_NBLK - 1, 0, osem0)
    wait_out(0, osem0)
    wait_out(1, osem1)


def _sc_call(mailbox):
    mesh = plsc.VectorSubcoreMesh(core_axis_name="c", subcore_axis_name="s")
    return pl.kernel(
        _sc_body,
        out_type=jax.ShapeDtypeStruct((_SC_ROWS, _D), jnp.float32),
        mesh=mesh,
        scratch_types=[
            pltpu.VMEM((2 * _C, _M, _D), jnp.float32),
            pltpu.VMEM((2 * _C, _D), jnp.float32),
            pltpu.SemaphoreType.DMA,
            pltpu.SemaphoreType.DMA,
            pltpu.SemaphoreType.DMA,
            pltpu.SemaphoreType.DMA,
        ],
    )(mailbox)


def _tc_body(mail_ref, out_ref):
    out_ref[...] = jnp.max(mail_ref[...], axis=1)


def _tc_call(mailbox):
    return pl.pallas_call(
        _tc_body,
        grid=(_T // _TCB,),
        in_specs=[pl.BlockSpec((_TCB, _M, _D), lambda i: (i, 0, 0))],
        out_specs=pl.BlockSpec((_TCB, _D), lambda i: (i, 0)),
        out_shape=jax.ShapeDtypeStruct((_N, _D), mailbox.dtype),
    )(mailbox)


def kernel(mailbox):
    sc_out = _sc_call(mailbox)
    tc_out = _tc_call(mailbox)
    return lax.dynamic_update_slice(tc_out, sc_out, (_T, 0))


# speedup vs baseline: 4.3285x; 1.3633x over previous
import jax
import jax.numpy as jnp
from jax.experimental import pallas as pl

_BLK = 512


def _max_body(mail_ref, out_ref):
    out_ref[...] = jnp.max(mail_ref[...], axis=1)


def kernel(mailbox):
    n, m, d = mailbox.shape
    grid = (-(-n // _BLK),)
    return pl.pallas_call(
        _max_body,
        grid=grid,
        in_specs=[pl.BlockSpec((_BLK, m, d), lambda i: (i, 0, 0))],
        out_specs=pl.BlockSpec((_BLK, d), lambda i: (i, 0)),
        out_shape=jax.ShapeDtypeStruct((n, d), mailbox.dtype),
    )(mailbox)
